# Initial kernel scaffold; baseline (speedup 1.0000x reference)
import jax
import jax.numpy as jnp
from jax.experimental import pallas as pl


def kernel(features, segments):
    raise NotImplementedError("write your pallas kernel here")


# speedup vs baseline: 3.2871x; 3.2871x over previous
import functools

import jax
import jax.numpy as jnp
from jax import lax
from jax.experimental import pallas as pl
from jax.experimental.pallas import tpu as pltpu
from jax.experimental.pallas import tpu_sc as plsc

N_EDGES = 320000
D_FEAT = 128
N_SEG = 10000

NUM_CORES = 2
NUM_SUBCORES = 16
LANES = 16
VPR = D_FEAT // LANES

BLK = 64
NBLK = N_EDGES // BLK
JMAX = -(-NBLK // NUM_SUBCORES)
SPC = 5120
ACC_ROWS = SPC + 8
DUMP = SPC
ROWS_TILE = SPC // NUM_SUBCORES
CH = BLK


def _sc_segment_mean(features, seg32, bounds):
  mesh = plsc.VectorSubcoreMesh(core_axis_name="c", subcore_axis_name="s")

  @functools.partial(
      pl.kernel,
      out_type=jax.ShapeDtypeStruct((NUM_CORES * SPC, D_FEAT), jnp.float32),
      mesh=mesh,
      scratch_types=[
          pltpu.VMEM_SHARED((ACC_ROWS, D_FEAT), jnp.float32),
          pltpu.VMEM_SHARED((ACC_ROWS, D_FEAT), jnp.float32),
          pltpu.VMEM((BLK, D_FEAT), jnp.float32),
          pltpu.VMEM((BLK,), jnp.int32),
          pltpu.VMEM((BLK, D_FEAT), jnp.float32),
          pltpu.VMEM((4, LANES), jnp.int32),
      ],
  )
  def k(feat_hbm, seg_hbm, bounds_hbm, out_hbm, acc, cacc, feat_v, idx_v,
        ones_v, bounds_v):
    c = lax.axis_index("c")
    s = lax.axis_index("s")
    zeros16 = jnp.zeros((LANES,), jnp.float32)
    ones16 = jnp.ones((LANES,), jnp.float32)

    pltpu.sync_copy(bounds_hbm, bounds_v)

    def fill(i, _):
      r = i // VPR
      k8 = i % VPR
      feat_v[r, pl.ds(k8 * LANES, LANES)] = zeros16
      ones_v[r, pl.ds(k8 * LANES, LANES)] = ones16
      return 0
    lax.fori_loop(0, BLK * VPR, fill, 0)

    def zero_acc(kk, _):
      r0 = s * ROWS_TILE + kk * CH
      pltpu.sync_copy(feat_v, acc.at[pl.ds(r0, CH)])
      pltpu.sync_copy(feat_v, cacc.at[pl.ds(r0, CH)])
      return 0
    lax.fori_loop(0, ROWS_TILE // CH, zero_acc, 0)

    @pl.when(s == 0)
    def _():
      pltpu.sync_copy(feat_v.at[pl.ds(0, 8)], acc.at[pl.ds(SPC, 8)])
      pltpu.sync_copy(feat_v.at[pl.ds(0, 8)], cacc.at[pl.ds(SPC, 8)])

    plsc.subcore_barrier()

    blo = bounds_v[2 * c, pl.ds(0, LANES)][0]
    bhi = bounds_v[2 * c + 1, pl.ds(0, LANES)][0]
    id0 = c * SPC

    def body(j, _):
      blk = blo + s + j * NUM_SUBCORES

      @pl.when(blk < bhi)
      def _():
        e0 = pl.multiple_of(blk * BLK, BLK)
        pltpu.sync_copy(feat_hbm.at[pl.ds(e0, BLK)], feat_v)
        pltpu.sync_copy(seg_hbm.at[pl.ds(e0, BLK)], idx_v)
        for kk in range(BLK // LANES):
          v = idx_v[pl.ds(kk * LANES, LANES)] - id0
          ok = (v >= 0) & (v < SPC)
          idx_v[pl.ds(kk * LANES, LANES)] = jnp.where(ok, v, DUMP)
        pltpu.sync_copy(feat_v, acc.at[idx_v], add=True)
        pltpu.sync_copy(ones_v, cacc.at[idx_v], add=True)
      return 0
    lax.fori_loop(0, JMAX, body, 0)

    plsc.subcore_barrier()

    def writeout(kk, _):
      r0 = s * ROWS_TILE + kk * CH
      pltpu.sync_copy(acc.at[pl.ds(r0, CH)], feat_v)
      pltpu.sync_copy(cacc.at[pl.ds(r0, CH)], ones_v)

      def div_row(i, _):
        r = i // VPR
        k8 = i % VPR
        sl = pl.ds(k8 * LANES, LANES)
        cnt = ones_v[r, sl]
        val = feat_v[r, sl] / jnp.maximum(cnt, 1.0)
        feat_v[r, sl] = jnp.where(cnt > 0.0, val, 0.0)
        return 0
      lax.fori_loop(0, CH * VPR, div_row, 0)

      pltpu.sync_copy(feat_v, out_hbm.at[pl.ds(c * SPC + r0, CH)])
      return 0
    lax.fori_loop(0, ROWS_TILE // CH, writeout, 0)


  return k(features, seg32, bounds)


@jax.jit
def kernel(features, segments):
  seg32 = segments.astype(jnp.int32)
  e0 = jnp.searchsorted(seg32, SPC).astype(jnp.int32)
  bounds = jnp.stack(
      [jnp.int32(0), (e0 + BLK - 1) // BLK, e0 // BLK, jnp.int32(NBLK)])
  bounds = jnp.broadcast_to(bounds[:, None], (4, LANES)).astype(jnp.int32)
  out = _sc_segment_mean(features, seg32, bounds)
  return out[:N_SEG]


# speedup vs baseline: 4.0551x; 1.2336x over previous
import functools

import jax
import jax.numpy as jnp
from jax import lax
from jax.experimental import pallas as pl
from jax.experimental.pallas import tpu as pltpu
from jax.experimental.pallas import tpu_sc as plsc

N_EDGES = 320000
D_FEAT = 128
N_SEG = 10000

NUM_CORES = 2
NUM_SUBCORES = 16
LANES = 16
VPR = D_FEAT // LANES

BLK = 128
NBLK = N_EDGES // BLK
JMAX = -(-NBLK // NUM_SUBCORES)
SPC = 5120
ACC_ROWS = SPC + 8
DUMP = SPC
ROWS_TILE = SPC // NUM_SUBCORES
CH = 64


def _sc_segment_mean(features, seg32, bounds):
  mesh = plsc.VectorSubcoreMesh(core_axis_name="c", subcore_axis_name="s")

  @functools.partial(
      pl.kernel,
      out_type=jax.ShapeDtypeStruct((NUM_CORES * SPC, D_FEAT), jnp.float32),
      mesh=mesh,
      scratch_types=[
          pltpu.VMEM_SHARED((ACC_ROWS, D_FEAT), jnp.float32),
          pltpu.VMEM_SHARED((ACC_ROWS, D_FEAT), jnp.float32),
          pltpu.VMEM((BLK, D_FEAT), jnp.float32),
          pltpu.VMEM((BLK,), jnp.int32),
          pltpu.VMEM((BLK, D_FEAT), jnp.float32),
          pltpu.VMEM((4, LANES), jnp.int32),
      ],
  )
  def k(feat_hbm, seg_hbm, bounds_hbm, out_hbm, acc, cacc, feat_v, idx_v,
        ones_v, bounds_v):
    c = lax.axis_index("c")
    s = lax.axis_index("s")
    zeros16 = jnp.zeros((LANES,), jnp.float32)
    ones16 = jnp.ones((LANES,), jnp.float32)

    pltpu.sync_copy(bounds_hbm, bounds_v)

    def fill(i, _):
      r = i // VPR
      k8 = i % VPR
      feat_v[r, pl.ds(k8 * LANES, LANES)] = zeros16
      ones_v[r, pl.ds(k8 * LANES, LANES)] = ones16
      return 0
    lax.fori_loop(0, BLK * VPR, fill, 0)

    def zero_acc(kk, _):
      r0 = s * ROWS_TILE + kk * CH
      pltpu.sync_copy(feat_v.at[pl.ds(0, CH)], acc.at[pl.ds(r0, CH)])
      pltpu.sync_copy(feat_v.at[pl.ds(0, CH)], cacc.at[pl.ds(r0, CH)])
      return 0
    lax.fori_loop(0, ROWS_TILE // CH, zero_acc, 0)

    @pl.when(s == 0)
    def _():
      pltpu.sync_copy(feat_v.at[pl.ds(0, 8)], acc.at[pl.ds(SPC, 8)])
      pltpu.sync_copy(feat_v.at[pl.ds(0, 8)], cacc.at[pl.ds(SPC, 8)])

    plsc.subcore_barrier()

    blo = bounds_v[2 * c, pl.ds(0, LANES)][0]
    bhi = bounds_v[2 * c + 1, pl.ds(0, LANES)][0]
    id0 = c * SPC

    def body(j, _):
      blk = blo + s + j * NUM_SUBCORES

      @pl.when(blk < bhi)
      def _():
        e0 = pl.multiple_of(blk * BLK, BLK)
        pltpu.sync_copy(feat_hbm.at[pl.ds(e0, BLK)], feat_v)
        pltpu.sync_copy(seg_hbm.at[pl.ds(e0, BLK)], idx_v)
        for kk in range(BLK // LANES):
          v = idx_v[pl.ds(kk * LANES, LANES)] - id0
          ok = (v >= 0) & (v < SPC)
          idx_v[pl.ds(kk * LANES, LANES)] = jnp.where(ok, v, DUMP)
        pltpu.sync_copy(feat_v, acc.at[idx_v], add=True)
        pltpu.sync_copy(ones_v, cacc.at[idx_v], add=True)
      return 0
    lax.fori_loop(0, JMAX, body, 0)

    plsc.subcore_barrier()

    def writeout(kk, _):
      r0 = s * ROWS_TILE + kk * CH
      pltpu.sync_copy(acc.at[pl.ds(r0, CH)], feat_v.at[pl.ds(0, CH)])
      pltpu.sync_copy(cacc.at[pl.ds(r0, CH)], ones_v.at[pl.ds(0, CH)])

      def div_row(i, _):
        r = i // VPR
        k8 = i % VPR
        sl = pl.ds(k8 * LANES, LANES)
        cnt = ones_v[r, sl]
        val = feat_v[r, sl] / jnp.maximum(cnt, 1.0)
        feat_v[r, sl] = jnp.where(cnt > 0.0, val, 0.0)
        return 0
      lax.fori_loop(0, CH * VPR, div_row, 0)

      pltpu.sync_copy(feat_v.at[pl.ds(0, CH)], out_hbm.at[pl.ds(c * SPC + r0, CH)])
      return 0
    lax.fori_loop(0, ROWS_TILE // CH, writeout, 0)


  return k(features, seg32, bounds)


@jax.jit
def kernel(features, segments):
  seg32 = segments.astype(jnp.int32)
  e0 = jnp.searchsorted(seg32, SPC).astype(jnp.int32)
  bounds = jnp.stack(
      [jnp.int32(0), (e0 + BLK - 1) // BLK, e0 // BLK, jnp.int32(NBLK)])
  bounds = jnp.broadcast_to(bounds[:, None], (4, LANES)).astype(jnp.int32)
  out = _sc_segment_mean(features, seg32, bounds)
  return out[:N_SEG]


# speedup vs baseline: 4.1891x; 1.0330x over previous
import functools

import jax
import jax.numpy as jnp
from jax import lax
from jax.experimental import pallas as pl
from jax.experimental.pallas import tpu as pltpu
from jax.experimental.pallas import tpu_sc as plsc

N_EDGES = 320000
D_FEAT = 128
N_SEG = 10000

NUM_CORES = 2
NUM_SUBCORES = 16
LANES = 16
VPR = D_FEAT // LANES

BLK = 128
NBLK = N_EDGES // BLK
JMAX = -(-NBLK // NUM_SUBCORES)
SPC = 5120
ACC_ROWS = SPC + 8
DUMP = SPC
ROWS_TILE = SPC // NUM_SUBCORES
CH = 64


def _sc_segment_mean(features, seg32, bounds):
  mesh = plsc.VectorSubcoreMesh(core_axis_name="c", subcore_axis_name="s")

  @functools.partial(
      pl.kernel,
      out_type=jax.ShapeDtypeStruct((NUM_CORES * SPC, D_FEAT), jnp.float32),
      mesh=mesh,
      scratch_types=[
          pltpu.VMEM_SHARED((ACC_ROWS, D_FEAT), jnp.float32),
          pltpu.VMEM_SHARED((ACC_ROWS, D_FEAT), jnp.float32),
          pltpu.VMEM((BLK, D_FEAT), jnp.float32),
          pltpu.VMEM((BLK,), jnp.int32),
          pltpu.VMEM((BLK, D_FEAT), jnp.float32),
          pltpu.VMEM((4, LANES), jnp.int32),
      ],
  )
  def k(feat_hbm, seg_hbm, bounds_hbm, out_hbm, acc, cacc,
        feat_v, idx_v, ones_v, bounds_v):
    c = lax.axis_index("c")
    s = lax.axis_index("s")
    zeros16 = jnp.zeros((LANES,), jnp.float32)
    ones16 = jnp.ones((LANES,), jnp.float32)

    pltpu.sync_copy(bounds_hbm, bounds_v)

    def fill(i, _):
      r = i // VPR
      k8 = i % VPR
      feat_v[r, pl.ds(k8 * LANES, LANES)] = zeros16
      ones_v[r, pl.ds(k8 * LANES, LANES)] = ones16
      return 0
    lax.fori_loop(0, BLK * VPR, fill, 0)

    def zero_acc(kk, _):
      r0 = s * ROWS_TILE + kk * CH
      pltpu.sync_copy(feat_v.at[pl.ds(0, CH)], acc.at[pl.ds(r0, CH)])
      pltpu.sync_copy(feat_v.at[pl.ds(0, CH)], cacc.at[pl.ds(r0, CH)])
      return 0
    lax.fori_loop(0, ROWS_TILE // CH, zero_acc, 0)

    @pl.when(s == 0)
    def _():
      pltpu.sync_copy(feat_v.at[pl.ds(0, 8)], acc.at[pl.ds(SPC, 8)])
      pltpu.sync_copy(feat_v.at[pl.ds(0, 8)], cacc.at[pl.ds(SPC, 8)])

    plsc.subcore_barrier()

    blo = bounds_v[2 * c, pl.ds(0, LANES)][0]
    bhi = bounds_v[2 * c + 1, pl.ds(0, LANES)][0]
    id0 = c * SPC

    def body(j, _):
      blk = blo + s + j * NUM_SUBCORES

      @pl.when(blk < bhi)
      def _():
        e0 = pl.multiple_of(blk * BLK, BLK)
        pltpu.sync_copy(feat_hbm.at[pl.ds(e0, BLK)], feat_v)
        pltpu.sync_copy(seg_hbm.at[pl.ds(e0, BLK)], idx_v)
        for kk in range(BLK // LANES):
          v = idx_v[pl.ds(kk * LANES, LANES)] - id0
          ok = (v >= 0) & (v < SPC)
          idx_v[pl.ds(kk * LANES, LANES)] = jnp.where(ok, v, DUMP)
        pltpu.sync_copy(feat_v, acc.at[idx_v], add=True)
        pltpu.sync_copy(ones_v, cacc.at[idx_v], add=True)
      return 0
    lax.fori_loop(0, JMAX, body, 0)

    plsc.subcore_barrier()

    def writeout(kk, _):
      r0 = s * ROWS_TILE + kk * CH
      pltpu.sync_copy(acc.at[pl.ds(r0, CH)], feat_v.at[pl.ds(0, CH)])
      pltpu.sync_copy(cacc.at[pl.ds(r0, CH)], ones_v.at[pl.ds(0, CH)])

      def div_row(r, _):
        cnt = ones_v[r, pl.ds(0, LANES)]
        pick = cnt > 0.0
        d = jnp.maximum(cnt, 1.0)
        for k8 in range(VPR):
          sl = pl.ds(k8 * LANES, LANES)
          feat_v[r, sl] = jnp.where(pick, feat_v[r, sl] / d, 0.0)
        return 0
      lax.fori_loop(0, CH, div_row, 0)

      pltpu.sync_copy(feat_v.at[pl.ds(0, CH)],
                      out_hbm.at[pl.ds(c * SPC + r0, CH)])
      return 0
    lax.fori_loop(0, ROWS_TILE // CH, writeout, 0)

  return k(features, seg32, bounds)


@jax.jit
def kernel(features, segments):
  seg32 = segments.astype(jnp.int32)
  e0 = jnp.searchsorted(seg32, SPC).astype(jnp.int32)
  bounds = jnp.stack(
      [jnp.int32(0), (e0 + BLK - 1) // BLK, e0 // BLK, jnp.int32(NBLK)])
  bounds = jnp.broadcast_to(bounds[:, None], (4, LANES)).astype(jnp.int32)
  out = _sc_segment_mean(features, seg32, bounds)
  return out[:N_SEG]


# speedup vs baseline: 4.2081x; 1.0045x over previous
import functools

import jax
import jax.numpy as jnp
from jax import lax
from jax.experimental import pallas as pl
from jax.experimental.pallas import tpu as pltpu
from jax.experimental.pallas import tpu_sc as plsc

N_EDGES = 320000
D_FEAT = 128
N_SEG = 10000

NUM_CORES = 2
NUM_SUBCORES = 16
LANES = 16
VPR = D_FEAT // LANES

BLK = 128
NBLK = N_EDGES // BLK
JMAX = -(-NBLK // NUM_SUBCORES)
SPC = 5120
ACC_ROWS = SPC + 8
DUMP = SPC
ROWS_TILE = SPC // NUM_SUBCORES
CH = 64


def _sc_segment_mean(features, seg32, bounds):
  mesh = plsc.VectorSubcoreMesh(core_axis_name="c", subcore_axis_name="s")

  @functools.partial(
      pl.kernel,
      out_type=jax.ShapeDtypeStruct((NUM_CORES * SPC, D_FEAT), jnp.float32),
      mesh=mesh,
      scratch_types=[
          pltpu.VMEM_SHARED((ACC_ROWS, D_FEAT), jnp.float32),
          pltpu.VMEM_SHARED((ACC_ROWS, D_FEAT), jnp.float32),
          pltpu.VMEM((BLK, D_FEAT), jnp.float32),
          pltpu.VMEM((BLK,), jnp.int32),
          pltpu.VMEM((BLK, D_FEAT), jnp.float32),
          pltpu.VMEM((4, LANES), jnp.int32),
          pltpu.SemaphoreType.DMA,
      ],
  )
  def k(feat_hbm, seg_hbm, bounds_hbm, out_hbm, acc, cacc,
        feat_v, idx_v, ones_v, bounds_v, csem):
    c = lax.axis_index("c")
    s = lax.axis_index("s")
    zeros16 = jnp.zeros((LANES,), jnp.float32)
    ones16 = jnp.ones((LANES,), jnp.float32)

    pltpu.sync_copy(bounds_hbm, bounds_v)

    def fill(i, _):
      r = i // VPR
      k8 = i % VPR
      feat_v[r, pl.ds(k8 * LANES, LANES)] = zeros16
      ones_v[r, pl.ds(k8 * LANES, LANES)] = ones16
      return 0
    lax.fori_loop(0, BLK * VPR, fill, 0)

    def zero_acc(kk, _):
      r0 = s * ROWS_TILE + kk * CH
      pltpu.sync_copy(feat_v.at[pl.ds(0, CH)], acc.at[pl.ds(r0, CH)])
      pltpu.sync_copy(feat_v.at[pl.ds(0, CH)], cacc.at[pl.ds(r0, CH)])
      return 0
    lax.fori_loop(0, ROWS_TILE // CH, zero_acc, 0)

    @pl.when(s == 0)
    def _():
      pltpu.sync_copy(feat_v.at[pl.ds(0, 8)], acc.at[pl.ds(SPC, 8)])
      pltpu.sync_copy(feat_v.at[pl.ds(0, 8)], cacc.at[pl.ds(SPC, 8)])

    plsc.subcore_barrier()

    blo = bounds_v[2 * c, pl.ds(0, LANES)][0]
    bhi = bounds_v[2 * c + 1, pl.ds(0, LANES)][0]
    id0 = c * SPC

    def body(j, _):
      blk = blo + s + j * NUM_SUBCORES

      @pl.when(blk < bhi)
      def _():
        e0 = pl.multiple_of(blk * BLK, BLK)
        pltpu.sync_copy(feat_hbm.at[pl.ds(e0, BLK)], feat_v)
        pltpu.sync_copy(seg_hbm.at[pl.ds(e0, BLK)], idx_v)
        for kk in range(BLK // LANES):
          v = idx_v[pl.ds(kk * LANES, LANES)] - id0
          ok = (v >= 0) & (v < SPC)
          idx_v[pl.ds(kk * LANES, LANES)] = jnp.where(ok, v, DUMP)
        cdma = pltpu.async_copy(ones_v, cacc.at[idx_v], csem, add=True)
        pltpu.sync_copy(feat_v, acc.at[idx_v], add=True)
        cdma.wait()
      return 0
    lax.fori_loop(0, JMAX, body, 0)

    plsc.subcore_barrier()

    def writeout(kk, _):
      r0 = s * ROWS_TILE + kk * CH
      pltpu.sync_copy(acc.at[pl.ds(r0, CH)], feat_v.at[pl.ds(0, CH)])
      pltpu.sync_copy(cacc.at[pl.ds(r0, CH)], ones_v.at[pl.ds(0, CH)])

      def div_row(r, _):
        cnt = ones_v[r, pl.ds(0, LANES)]
        pick = cnt > 0.0
        d = jnp.maximum(cnt, 1.0)
        for k8 in range(VPR):
          sl = pl.ds(k8 * LANES, LANES)
          feat_v[r, sl] = jnp.where(pick, feat_v[r, sl] / d, 0.0)
        return 0
      lax.fori_loop(0, CH, div_row, 0)

      pltpu.sync_copy(feat_v.at[pl.ds(0, CH)],
                      out_hbm.at[pl.ds(c * SPC + r0, CH)])
      return 0
    lax.fori_loop(0, ROWS_TILE // CH, writeout, 0)

  return k(features, seg32, bounds)


@jax.jit
def kernel(features, segments):
  seg32 = segments.astype(jnp.int32)
  e0 = jnp.searchsorted(seg32, SPC).astype(jnp.int32)
  bounds = jnp.stack(
      [jnp.int32(0), (e0 + BLK - 1) // BLK, e0 // BLK, jnp.int32(NBLK)])
  bounds = jnp.broadcast_to(bounds[:, None], (4, LANES)).astype(jnp.int32)
  out = _sc_segment_mean(features, seg32, bounds)
  return out[:N_SEG]


# speedup vs baseline: 5.3715x; 1.2765x over previous
import functools

import jax
import jax.numpy as jnp
from jax import lax
from jax.experimental import pallas as pl
from jax.experimental.pallas import tpu as pltpu
from jax.experimental.pallas import tpu_sc as plsc

N_EDGES = 320000
D_FEAT = 128
N_SEG = 10000

NUM_CORES = 2
NUM_SUBCORES = 16
LANES = 16
VPR = D_FEAT // LANES

BLK = 128
HALF = BLK // 2
NBLK = N_EDGES // BLK
JMAX = -(-NBLK // NUM_SUBCORES)
JMAX2 = -(-JMAX // 2)
SPC = 5120
ACC_ROWS = SPC + 8
DUMP = SPC
ROWS_TILE = SPC // NUM_SUBCORES
CH = 64


def _sc_segment_mean(features, seg32, bounds):
  mesh = plsc.VectorSubcoreMesh(core_axis_name="c", subcore_axis_name="s")

  @functools.partial(
      pl.kernel,
      out_type=jax.ShapeDtypeStruct((NUM_CORES * SPC, D_FEAT), jnp.float32),
      mesh=mesh,
      scratch_types=[
          pltpu.VMEM_SHARED((ACC_ROWS, D_FEAT), jnp.float32),
          pltpu.VMEM_SHARED((ACC_ROWS, D_FEAT), jnp.float32),
          pltpu.VMEM((BLK, D_FEAT), jnp.float32),
          pltpu.VMEM((BLK, D_FEAT), jnp.float32),
          pltpu.VMEM((HALF, D_FEAT), jnp.float32),
          pltpu.VMEM((BLK,), jnp.int32),
          pltpu.VMEM((2, HALF), jnp.int32),
          pltpu.VMEM((4, LANES), jnp.int32),
          pltpu.SemaphoreType.DMA,
          pltpu.SemaphoreType.DMA,
          pltpu.SemaphoreType.DMA,
      ],
  )
  def k(feat_hbm, seg_hbm, bounds_hbm, out_hbm, acc, cacc,
        feat_a, feat_b, ones_v, idx_v, idx2_v, bounds_v, sem_a, sem_b, csem):
    c = lax.axis_index("c")
    s = lax.axis_index("s")
    zeros16 = jnp.zeros((LANES,), jnp.float32)
    ones16 = jnp.ones((LANES,), jnp.float32)

    pltpu.sync_copy(bounds_hbm, bounds_v)

    def fill(i, _):
      r = i // VPR
      k8 = i % VPR
      feat_a[r, pl.ds(k8 * LANES, LANES)] = zeros16
      return 0
    lax.fori_loop(0, BLK * VPR, fill, 0)

    def fill1(i, _):
      r = i // VPR
      k8 = i % VPR
      ones_v[r, pl.ds(k8 * LANES, LANES)] = ones16
      return 0
    lax.fori_loop(0, HALF * VPR, fill1, 0)

    def zero_acc(kk, _):
      r0 = s * ROWS_TILE + kk * CH
      pltpu.sync_copy(feat_a.at[pl.ds(0, CH)], acc.at[pl.ds(r0, CH)])
      pltpu.sync_copy(feat_a.at[pl.ds(0, CH)], cacc.at[pl.ds(r0, CH)])
      return 0
    lax.fori_loop(0, ROWS_TILE // CH, zero_acc, 0)

    @pl.when(s == 0)
    def _():
      pltpu.sync_copy(feat_a.at[pl.ds(0, 8)], acc.at[pl.ds(SPC, 8)])
      pltpu.sync_copy(feat_a.at[pl.ds(0, 8)], cacc.at[pl.ds(SPC, 8)])

    plsc.subcore_barrier()

    blo = bounds_v[2 * c, pl.ds(0, LANES)][0]
    bhi = bounds_v[2 * c + 1, pl.ds(0, LANES)][0]
    id0 = c * SPC

    def start_load(blk, fv, sem):
      e0 = pl.multiple_of(blk * BLK, BLK)
      pltpu.async_copy(feat_hbm.at[pl.ds(e0, BLK)], fv, sem)

    def wait_load(blk, fv, sem):
      e0 = pl.multiple_of(blk * BLK, BLK)
      pltpu.make_async_copy(feat_hbm.at[pl.ds(e0, BLK)], fv, sem).wait()

    blk0 = blo + s

    @pl.when(blk0 < bhi)
    def _():
      start_load(blk0, feat_a, sem_a)

    bufs = ((feat_a, sem_a), (feat_b, sem_b))

    def body(j2, _):
      for p in range(2):
        fv, sem = bufs[p]
        nfv, nsem = bufs[1 - p]
        blk = blo + s + (j2 * 2 + p) * NUM_SUBCORES
        nxt = blk + NUM_SUBCORES

        @pl.when(nxt < bhi)
        def _():
          start_load(nxt, nfv, nsem)

        @pl.when(blk < bhi)
        def _():
          e0 = pl.multiple_of(blk * BLK, BLK)
          pltpu.sync_copy(seg_hbm.at[pl.ds(e0, BLK)], idx_v)
          for kk in range(BLK // LANES):
            v = idx_v[pl.ds(kk * LANES, LANES)] - id0
            ok = (v >= 0) & (v < SPC)
            w = jnp.where(ok, v, DUMP)
            idx_v[pl.ds(kk * LANES, LANES)] = w
            idx2_v[kk // (HALF // LANES),
                   pl.ds((kk % (HALF // LANES)) * LANES, LANES)] = w
          wait_load(blk, fv, sem)
          c0 = pltpu.async_copy(ones_v, cacc.at[idx2_v.at[0]], csem, add=True)
          c1 = pltpu.async_copy(ones_v, cacc.at[idx2_v.at[1]], csem, add=True)
          pltpu.sync_copy(fv, acc.at[idx_v], add=True)
          c0.wait()
          c1.wait()
      return 0
    lax.fori_loop(0, JMAX2, body, 0)

    plsc.subcore_barrier()

    def writeout(kk, _):
      r0 = s * ROWS_TILE + kk * CH
      pltpu.sync_copy(acc.at[pl.ds(r0, CH)], feat_a.at[pl.ds(0, CH)])
      pltpu.sync_copy(cacc.at[pl.ds(r0, CH)], feat_b.at[pl.ds(0, CH)])

      def div_row(r, _):
        cnt = feat_b[r, pl.ds(0, LANES)]
        pick = cnt > 0.0
        d = jnp.maximum(cnt, 1.0)
        for k8 in range(VPR):
          sl = pl.ds(k8 * LANES, LANES)
          feat_a[r, sl] = jnp.where(pick, feat_a[r, sl] / d, 0.0)
        return 0
      lax.fori_loop(0, CH, div_row, 0)

      pltpu.sync_copy(feat_a.at[pl.ds(0, CH)],
                      out_hbm.at[pl.ds(c * SPC + r0, CH)])
      return 0
    lax.fori_loop(0, ROWS_TILE // CH, writeout, 0)

  return k(features, seg32, bounds)


@jax.jit
def kernel(features, segments):
  seg32 = segments.astype(jnp.int32)
  e0 = jnp.searchsorted(seg32, SPC).astype(jnp.int32)
  bounds = jnp.stack(
      [jnp.int32(0), (e0 + BLK - 1) // BLK, e0 // BLK, jnp.int32(NBLK)])
  bounds = jnp.broadcast_to(bounds[:, None], (4, LANES)).astype(jnp.int32)
  out = _sc_segment_mean(features, seg32, bounds)
  return out[:N_SEG]
